# baseline (speedup 1.0000x reference)
import functools

import jax
import jax.numpy as jnp
from jax import lax
from jax.experimental import pallas as pl
from jax.experimental.pallas import tpu as pltpu

_B = 64
_S = 576
_D = 768
_KC = 12
_CR = _S // _KC
_NSTEP = _B * _KC


def _tc_body(in_ref, posm_hbm, cls_hbm, pos0_hbm, out_hbm,
             pos_v, cls_v, t0_v, t1_v, ob0, ob1, psem, out_sems, cls_sems):
    b = pl.program_id(0)
    k = pl.program_id(1)
    step = b * _KC + k

    @pl.when(step == 0)
    def _():
        pltpu.async_copy(posm_hbm, pos_v, psem).wait()
        pltpu.async_copy(cls_hbm, t0_v, psem).wait()
        pltpu.async_copy(pos0_hbm, t1_v, psem).wait()
        cls_v[...] = t0_v[...] + t1_v[...]

    def out_dst(kk, bb):
        return out_hbm.at[pl.ds(1 + kk * _CR, _CR), bb, :]

    @pl.when(step >= 2)
    def _():
        @pl.when(k % 2 == 0)
        def _():
            pltpu.make_async_copy(ob0, out_dst(0, 0), out_sems.at[0]).wait()

        @pl.when(k % 2 == 1)
        def _():
            pltpu.make_async_copy(ob1, out_dst(0, 0), out_sems.at[1]).wait()

    @pl.when(k % 2 == 0)
    def _():
        ob0[...] = in_ref[0] + pos_v[pl.ds(k * _CR, _CR), :]
        pltpu.async_copy(ob0, out_dst(k, b), out_sems.at[0])

    @pl.when(k % 2 == 1)
    def _():
        ob1[...] = in_ref[0] + pos_v[pl.ds(k * _CR, _CR), :]
        pltpu.async_copy(ob1, out_dst(k, b), out_sems.at[1])

    @pl.when(k == 0)
    def _():
        @pl.when(b >= 2)
        def _():
            pltpu.make_async_copy(
                cls_v, out_hbm.at[pl.ds(0, 1), 0, :], cls_sems.at[b % 2]).wait()

        pltpu.async_copy(cls_v, out_hbm.at[pl.ds(0, 1), b, :],
                         cls_sems.at[b % 2])

    @pl.when(step == _NSTEP - 1)
    def _():
        pltpu.make_async_copy(ob0, out_dst(0, 0), out_sems.at[0]).wait()
        pltpu.make_async_copy(ob1, out_dst(0, 0), out_sems.at[1]).wait()
        pltpu.make_async_copy(
            cls_v, out_hbm.at[pl.ds(0, 1), 0, :], cls_sems.at[0]).wait()
        pltpu.make_async_copy(
            cls_v, out_hbm.at[pl.ds(0, 1), 0, :], cls_sems.at[1]).wait()


@jax.jit
def _run_tc(inputs, class_embed, position_table):
    pos0 = lax.slice(position_table, (0, 0), (1, _D))
    posm = lax.slice(position_table, (1, 0), (_S + 1, _D))
    cls2 = class_embed.reshape(1, _D)
    out_t = pl.pallas_call(
        _tc_body,
        grid=(_B, _KC),
        in_specs=[
            pl.BlockSpec((1, _CR, _D), lambda b, k: (b, k, 0)),
            pl.BlockSpec(memory_space=pltpu.HBM),
            pl.BlockSpec(memory_space=pltpu.HBM),
            pl.BlockSpec(memory_space=pltpu.HBM),
        ],
        out_specs=pl.BlockSpec(memory_space=pltpu.HBM),
        out_shape=jax.ShapeDtypeStruct((_S + 1, _B, _D), jnp.float32),
        scratch_shapes=[
            pltpu.VMEM((_S, _D), jnp.float32),
            pltpu.VMEM((1, _D), jnp.float32),
            pltpu.VMEM((1, _D), jnp.float32),
            pltpu.VMEM((1, _D), jnp.float32),
            pltpu.VMEM((_CR, _D), jnp.float32),
            pltpu.VMEM((_CR, _D), jnp.float32),
            pltpu.SemaphoreType.DMA,
            pltpu.SemaphoreType.DMA((2,)),
            pltpu.SemaphoreType.DMA((2,)),
        ],
    )(inputs, posm, cls2, pos0)
    return out_t


def kernel(inputs, class_embed, position_table):
    out_t = _run_tc(inputs, class_embed, position_table)
    return out_t.transpose(1, 0, 2)


# speedup vs baseline: 4.5746x; 4.5746x over previous
import functools

import jax
import jax.numpy as jnp
from jax import lax
from jax.experimental import pallas as pl
from jax.experimental.pallas import tpu as pltpu

_B = 64
_S = 576
_D = 768
_SB = 8
_NBLK = (_S + 1 + _SB - 1) // _SB
_NBUF = 3


def _in_dmas(in_hbm, buf, sem, j):
    for s in range(_SB):
        row = jnp.clip(_SB * j + s - 1, 0, _S - 1)
        pltpu.async_copy(in_hbm.at[:, row, :], buf.at[s], sem)


def _wait_in(in_hbm, buf, sem):
    for s in range(_SB):
        pltpu.make_async_copy(in_hbm.at[:, 0, :], buf.at[s], sem).wait()


def _tc_body(in_hbm, pos_ref, cls_ref, out_ref, b0, b1, b2, in_sems):
    j = pl.program_id(0)
    bufs = (b0, b1, b2)

    @pl.when(j == 0)
    def _():
        _in_dmas(in_hbm, bufs[0], in_sems.at[0], 0)
        _in_dmas(in_hbm, bufs[1], in_sems.at[1], 1)

    for p in range(_NBUF):
        @pl.when(j % _NBUF == p)
        def _(p=p):
            _wait_in(in_hbm, bufs[p], in_sems.at[p])

            pos_b = jnp.broadcast_to(
                pos_ref[...].reshape(_SB, 1, _D), (_SB, _B, _D))
            out_ref[...] = bufs[p][...] + pos_b

            @pl.when(j + 2 < _NBLK)
            def _():
                q = (p + 2) % _NBUF
                _in_dmas(in_hbm, bufs[q], in_sems.at[q], j + 2)

    @pl.when(j == 0)
    def _():
        row0 = cls_ref[...] + pos_ref[0:1, :]
        out_ref[0:1, :, :] = jnp.broadcast_to(
            row0.reshape(1, 1, _D), (1, _B, _D))


@jax.jit
def _run_tc(inputs, class_embed, position_table):
    pos_pad = lax.slice(position_table, (0, 0), (_NBLK * _SB, _D))
    cls2 = class_embed.reshape(1, _D)
    out_t = pl.pallas_call(
        _tc_body,
        grid=(_NBLK,),
        in_specs=[
            pl.BlockSpec(memory_space=pltpu.HBM),
            pl.BlockSpec((_SB, _D), lambda j: (j, 0)),
            pl.BlockSpec((1, _D), lambda j: (0, 0)),
        ],
        out_specs=pl.BlockSpec((_SB, _B, _D), lambda j: (j, 0, 0)),
        out_shape=jax.ShapeDtypeStruct((_S + 1, _B, _D), jnp.float32),
        scratch_shapes=[
            pltpu.VMEM((_SB, _B, _D), jnp.float32),
            pltpu.VMEM((_SB, _B, _D), jnp.float32),
            pltpu.VMEM((_SB, _B, _D), jnp.float32),
            pltpu.SemaphoreType.DMA((_NBUF,)),
        ],
    )(inputs, pos_pad, cls2)
    return out_t


def kernel(inputs, class_embed, position_table):
    out_t = _run_tc(inputs, class_embed, position_table)
    return out_t.transpose(1, 0, 2)
